# Initial kernel scaffold; baseline (speedup 1.0000x reference)
import jax
import jax.numpy as jnp
from jax.experimental import pallas as pl


def kernel(x, flat, edge_index, edge_weight, W1, b1, W2, b2, Wf, bf, Wo, bo):
    raise NotImplementedError("write your pallas kernel here")


# speedup vs baseline: 7.5770x; 7.5770x over previous
import jax
import jax.numpy as jnp
from jax import lax
from jax.experimental import pallas as pl
from jax.experimental.pallas import tpu as pltpu
from jax.experimental.pallas import tpu_sc as plsc

N = 10000
E = 320000
D = 128
NC = 2
NS = 16
NW = NC * NS
CH = 128
NCH = 80
EPAD = NW * CH * NCH
NPAD = 10240
RPT = NPAD // NS

_MESH = plsc.VectorSubcoreMesh(core_axis_name="c", subcore_axis_name="s")


def _sc_deg_body(col2_hbm, ew2_hbm, zn_hbm, degp_hbm, col_v, ew_v, deg_sh):
    c = lax.axis_index("c")
    s = lax.axis_index("s")
    w = s * NC + c
    pltpu.sync_copy(col2_hbm.at[pl.ds(w * NCH, NCH)], col_v)
    pltpu.sync_copy(ew2_hbm.at[pl.ds(w * NCH, NCH)], ew_v)
    pltpu.sync_copy(zn_hbm.at[pl.ds(s * RPT, RPT)],
                    deg_sh.at[pl.ds(s * RPT, RPT)])
    plsc.subcore_barrier()

    def deg_chunk(j, carry):
        pltpu.sync_copy(ew_v.at[j], deg_sh.at[col_v.at[j]], add=True)
        return carry

    lax.fori_loop(0, NCH, deg_chunk, 0)
    plsc.subcore_barrier()
    pltpu.sync_copy(deg_sh.at[pl.ds(s * RPT, RPT)],
                    degp_hbm.at[c, pl.ds(s * RPT, RPT)])


_sc_deg = pl.kernel(
    _sc_deg_body,
    out_type=jax.ShapeDtypeStruct((NC, NPAD), jnp.float32),
    mesh=_MESH,
    compiler_params=pltpu.CompilerParams(needs_layout_passes=False),
    scratch_types=(
        pltpu.VMEM((NCH, CH), jnp.int32),
        pltpu.VMEM((NCH, CH), jnp.float32),
        pltpu.VMEM_SHARED((NPAD,), jnp.float32),
    ),
)


BLK = 16
NBLK = NCH // BLK


def _sc_prop_body(table_hbm, row2_hbm, col2_hbm, ew2_hbm, z2_hbm, dis_hbm,
                  parts_hbm,
                  col_v, ew_v, row_v, dis_v, w_v, msg_v, acc_sh, sem):
    c = lax.axis_index("c")
    s = lax.axis_index("s")
    w = s * NC + c

    pltpu.sync_copy(dis_hbm, dis_v)
    pltpu.sync_copy(z2_hbm.at[pl.ds(s * RPT, RPT)],
                    acc_sh.at[pl.ds(s * RPT, RPT)])
    plsc.subcore_barrier()

    def block_body(b, carry):
        base = w * NCH + b * BLK
        pltpu.sync_copy(col2_hbm.at[pl.ds(base, BLK)], col_v)
        pltpu.sync_copy(ew2_hbm.at[pl.ds(base, BLK)], ew_v)
        pltpu.sync_copy(row2_hbm.at[pl.ds(base, BLK)], row_v)

        def chunk_body(j, carry2):
            gat = pltpu.async_copy(table_hbm.at[row_v.at[j]], msg_v, sem)
            for g in range(8):
                r16 = row_v[j, pl.ds(g * 16, 16)]
                d16 = plsc.load_gather(dis_v, [r16])
                e16 = ew_v[j, pl.ds(g * 16, 16)]
                w_v[pl.ds(g * 16, 16)] = e16 * d16
            gat.wait()

            def edge_body(e, carry3):
                spl = plsc.load_gather(w_v, [jnp.zeros((16,), jnp.int32) + e])
                for f in range(8):
                    sl = pl.ds(f * 16, 16)
                    msg_v[e, sl] = msg_v[e, sl] * spl
                return carry3

            lax.fori_loop(0, CH, edge_body, 0)
            pltpu.sync_copy(msg_v, acc_sh.at[col_v.at[j]], add=True)
            return carry2

        lax.fori_loop(0, BLK, chunk_body, 0)
        return carry

    lax.fori_loop(0, NBLK, block_body, 0)
    plsc.subcore_barrier()
    pltpu.sync_copy(acc_sh.at[pl.ds(s * RPT, RPT)],
                    parts_hbm.at[c, pl.ds(s * RPT, RPT)])


_sc_prop = pl.kernel(
    _sc_prop_body,
    out_type=jax.ShapeDtypeStruct((NC, NPAD, D), jnp.float32),
    mesh=_MESH,
    compiler_params=pltpu.CompilerParams(needs_layout_passes=False),
    scratch_types=(
        pltpu.VMEM((BLK, CH), jnp.int32),
        pltpu.VMEM((BLK, CH), jnp.float32),
        pltpu.VMEM((BLK, CH), jnp.int32),
        pltpu.VMEM((NPAD,), jnp.float32),
        pltpu.VMEM((CH,), jnp.float32),
        pltpu.VMEM((CH, D), jnp.float32),
        pltpu.VMEM_SHARED((NPAD, D), jnp.float32),
        pltpu.SemaphoreType.DMA,
    ),
)


def _first_body(x_ref, w1_ref, p0_ref, p1_ref, h1_ref, dis_ref):
    h1_ref[...] = jnp.dot(x_ref[...], w1_ref[...],
                          preferred_element_type=jnp.float32)
    dis_ref[...] = lax.rsqrt(1.0 + p0_ref[...] + p1_ref[...])


def _mid_body(r0_ref, r1_ref, h1_ref, dis_ref, b1_ref, w2_ref, o_ref):
    dis = dis_ref[...]
    t = dis * (r0_ref[...] + r1_ref[...] + dis * h1_ref[...]) + b1_ref[...]
    t = jnp.maximum(t, 0.0)
    o_ref[...] = jnp.dot(t, w2_ref[...], preferred_element_type=jnp.float32)


def _head_body(r0_ref, r1_ref, h2_ref, dis_ref, b2_ref, flat_ref, wf_ref,
               bf_ref, woh_ref, wof_ref, bo_ref, o_ref):
    dis = dis_ref[...]
    z = dis * (r0_ref[...] + r1_ref[...] + dis * h2_ref[...]) + b2_ref[...]
    f = jnp.dot(flat_ref[...], wf_ref[...],
                preferred_element_type=jnp.float32) + bf_ref[...]
    out = (jnp.dot(z, woh_ref[...], preferred_element_type=jnp.float32)
           + jnp.dot(f, wof_ref[...], preferred_element_type=jnp.float32)
           + bo_ref[...])
    o_ref[...] = jnp.maximum(out, 0.0)


def kernel(x, flat, edge_index, edge_weight, W1, b1, W2, b2, Wf, bf, Wo, bo):
    row = edge_index[0].astype(jnp.int32)
    col = edge_index[1].astype(jnp.int32)
    ew = edge_weight.reshape(-1).astype(jnp.float32)
    pad = EPAD - E
    row2 = jnp.concatenate([row, jnp.zeros((pad,), jnp.int32)]).reshape(-1, CH)
    col2 = jnp.concatenate([col, jnp.zeros((pad,), jnp.int32)]).reshape(-1, CH)
    ew2 = jnp.concatenate([ew, jnp.zeros((pad,), jnp.float32)]).reshape(-1, CH)
    zeros2d = jnp.zeros((NPAD, D), jnp.float32)
    zerosn = jnp.zeros((NPAD,), jnp.float32)

    degp = _sc_deg(col2, ew2, zerosn)

    h1, dis80 = pl.pallas_call(
        _first_body,
        out_shape=(jax.ShapeDtypeStruct((N, D), jnp.float32),
                   jax.ShapeDtypeStruct((NPAD // CH, CH), jnp.float32)),
    )(x, W1, degp[0].reshape(NPAD // CH, CH), degp[1].reshape(NPAD // CH, CH))

    dis = dis80.reshape(NPAD)
    parts1 = _sc_prop(h1, row2, col2, ew2, zeros2d, dis)

    dis_col = dis[:N].reshape(N, 1)
    h2 = pl.pallas_call(
        _mid_body,
        out_shape=jax.ShapeDtypeStruct((N, D), jnp.float32),
    )(parts1[0, :N], parts1[1, :N], h1, dis_col, b1.reshape(1, D), W2)

    parts2 = _sc_prop(h2, row2, col2, ew2, zeros2d, dis)

    out = pl.pallas_call(
        _head_body,
        out_shape=jax.ShapeDtypeStruct((N, Wo.shape[1]), jnp.float32),
    )(parts2[0, :N], parts2[1, :N], h2, dis_col, b2.reshape(1, D),
      flat, Wf, bf.reshape(1, -1), Wo[:D], Wo[D:], bo.reshape(1, -1))
    return out


# speedup vs baseline: 9.8986x; 1.3064x over previous
import jax
import jax.numpy as jnp
from jax import lax
from jax.experimental import pallas as pl
from jax.experimental.pallas import tpu as pltpu
from jax.experimental.pallas import tpu_sc as plsc

N = 10000
E = 320000
D = 128
NC = 2
NS = 16
NW = NC * NS
CH = 128
NCH = 80
EPAD = NW * CH * NCH
NPAD = 10240
RPT = NPAD // NS
BLK = 16
NBLK = NCH // BLK

_MESH = plsc.VectorSubcoreMesh(core_axis_name="c", subcore_axis_name="s")


def _sc_deg_body(col2_hbm, ew2_hbm, zn_hbm, degp_hbm, col_v, ew_v, deg_sh):
    c = lax.axis_index("c")
    s = lax.axis_index("s")
    w = s * NC + c
    pltpu.sync_copy(col2_hbm.at[pl.ds(w * NCH, NCH)], col_v)
    pltpu.sync_copy(ew2_hbm.at[pl.ds(w * NCH, NCH)], ew_v)
    pltpu.sync_copy(zn_hbm.at[pl.ds(s * RPT, RPT)],
                    deg_sh.at[pl.ds(s * RPT, RPT)])
    plsc.subcore_barrier()

    def deg_chunk(j, carry):
        pltpu.sync_copy(ew_v.at[j], deg_sh.at[col_v.at[j]], add=True)
        return carry

    lax.fori_loop(0, NCH, deg_chunk, 0)
    plsc.subcore_barrier()
    pltpu.sync_copy(deg_sh.at[pl.ds(s * RPT, RPT)],
                    degp_hbm.at[c, pl.ds(s * RPT, RPT)])


_sc_deg = pl.kernel(
    _sc_deg_body,
    out_type=jax.ShapeDtypeStruct((NC, NPAD), jnp.float32),
    mesh=_MESH,
    compiler_params=pltpu.CompilerParams(needs_layout_passes=False),
    scratch_types=(
        pltpu.VMEM((NCH, CH), jnp.int32),
        pltpu.VMEM((NCH, CH), jnp.float32),
        pltpu.VMEM_SHARED((NPAD,), jnp.float32),
    ),
)


def _sc_prop_body(table_hbm, row2_hbm, col2_hbm, ew2_hbm, z2_hbm,
                  parts_hbm,
                  col_v, ew_v, row_v, w_v, msg0, msg1, acc_sh, sem0, sem1):
    c = lax.axis_index("c")
    s = lax.axis_index("s")
    w = s * NC + c

    pltpu.sync_copy(z2_hbm.at[pl.ds(s * RPT, RPT)],
                    acc_sh.at[pl.ds(s * RPT, RPT)])
    plsc.subcore_barrier()

    def scale_scatter(j, msg):
        for g in range(8):
            sl = pl.ds(g * 16, 16)
            w_v[sl] = ew_v[j, sl]

        def edge_body(e, carry3):
            spl = plsc.load_gather(w_v, [jnp.zeros((16,), jnp.int32) + e])
            for f in range(8):
                sl = pl.ds(f * 16, 16)
                msg[e, sl] = msg[e, sl] * spl
            return carry3

        lax.fori_loop(0, CH, edge_body, 0)
        pltpu.sync_copy(msg, acc_sh.at[col_v.at[j]], add=True)

    def block_body(b, carry):
        base = w * NCH + b * BLK
        pltpu.sync_copy(col2_hbm.at[pl.ds(base, BLK)], col_v)
        pltpu.sync_copy(ew2_hbm.at[pl.ds(base, BLK)], ew_v)
        pltpu.sync_copy(row2_hbm.at[pl.ds(base, BLK)], row_v)
        pltpu.async_copy(table_hbm.at[row_v.at[0]], msg0, sem0)

        def pair_body(p, carry2):
            ja = 2 * p
            jb = 2 * p + 1
            pltpu.make_async_copy(table_hbm.at[row_v.at[ja]], msg0, sem0).wait()
            pltpu.async_copy(table_hbm.at[row_v.at[jb]], msg1, sem1)
            scale_scatter(ja, msg0)
            pltpu.make_async_copy(table_hbm.at[row_v.at[jb]], msg1, sem1).wait()

            @pl.when(p < BLK // 2 - 1)
            def _():
                pltpu.async_copy(table_hbm.at[row_v.at[ja + 2]], msg0, sem0)

            scale_scatter(jb, msg1)
            return carry2

        lax.fori_loop(0, BLK // 2, pair_body, 0)
        return carry

    lax.fori_loop(0, NBLK, block_body, 0)
    plsc.subcore_barrier()
    pltpu.sync_copy(acc_sh.at[pl.ds(s * RPT, RPT)],
                    parts_hbm.at[c, pl.ds(s * RPT, RPT)])


_sc_prop = pl.kernel(
    _sc_prop_body,
    out_type=jax.ShapeDtypeStruct((NC, NPAD, D), jnp.float32),
    mesh=_MESH,
    compiler_params=pltpu.CompilerParams(needs_layout_passes=False),
    scratch_types=(
        pltpu.VMEM((BLK, CH), jnp.int32),
        pltpu.VMEM((BLK, CH), jnp.float32),
        pltpu.VMEM((BLK, CH), jnp.int32),
        pltpu.VMEM((CH,), jnp.float32),
        pltpu.VMEM((CH, D), jnp.float32),
        pltpu.VMEM((CH, D), jnp.float32),
        pltpu.VMEM_SHARED((NPAD, D), jnp.float32),
        pltpu.SemaphoreType.DMA,
        pltpu.SemaphoreType.DMA,
    ),
)


def _first_body(x_ref, w1_ref, p0_ref, p1_ref, g1_ref, dis_ref):
    dis = lax.rsqrt(1.0 + p0_ref[...] + p1_ref[...])
    dis_ref[...] = dis
    g1_ref[...] = dis[:N] * jnp.dot(x_ref[...], w1_ref[...],
                                    preferred_element_type=jnp.float32)


def _mid_body(r0_ref, r1_ref, g1_ref, dis_ref, b1_ref, w2_ref, o_ref):
    dis = dis_ref[...]
    t = dis * (r0_ref[...] + r1_ref[...] + g1_ref[...]) + b1_ref[...]
    t = jnp.maximum(t, 0.0)
    o_ref[...] = dis * jnp.dot(t, w2_ref[...],
                               preferred_element_type=jnp.float32)


def _head_body(r0_ref, r1_ref, g2_ref, dis_ref, b2_ref, flat_ref, wf_ref,
               bf_ref, woh_ref, wof_ref, bo_ref, o_ref):
    dis = dis_ref[...]
    z = dis * (r0_ref[...] + r1_ref[...] + g2_ref[...]) + b2_ref[...]
    f = jnp.dot(flat_ref[...], wf_ref[...],
                preferred_element_type=jnp.float32) + bf_ref[...]
    out = (jnp.dot(z, woh_ref[...], preferred_element_type=jnp.float32)
           + jnp.dot(f, wof_ref[...], preferred_element_type=jnp.float32)
           + bo_ref[...])
    o_ref[...] = jnp.maximum(out, 0.0)


def kernel(x, flat, edge_index, edge_weight, W1, b1, W2, b2, Wf, bf, Wo, bo):
    row = edge_index[0].astype(jnp.int32)
    col = edge_index[1].astype(jnp.int32)
    ew = edge_weight.reshape(-1).astype(jnp.float32)
    pad = EPAD - E
    row2 = jnp.concatenate([row, jnp.zeros((pad,), jnp.int32)]).reshape(-1, CH)
    col2 = jnp.concatenate([col, jnp.zeros((pad,), jnp.int32)]).reshape(-1, CH)
    ew2 = jnp.concatenate([ew, jnp.zeros((pad,), jnp.float32)]).reshape(-1, CH)
    zeros2d = jnp.zeros((NPAD, D), jnp.float32)
    zerosn = jnp.zeros((NPAD,), jnp.float32)

    degp = _sc_deg(col2, ew2, zerosn)

    g1, dis = pl.pallas_call(
        _first_body,
        out_shape=(jax.ShapeDtypeStruct((N, D), jnp.float32),
                   jax.ShapeDtypeStruct((NPAD, 1), jnp.float32)),
    )(x, W1, degp[0].reshape(NPAD, 1), degp[1].reshape(NPAD, 1))

    parts1 = _sc_prop(g1, row2, col2, ew2, zeros2d)

    dis_col = dis[:N]
    g2 = pl.pallas_call(
        _mid_body,
        out_shape=jax.ShapeDtypeStruct((N, D), jnp.float32),
    )(parts1[0, :N], parts1[1, :N], g1, dis_col, b1.reshape(1, D), W2)

    parts2 = _sc_prop(g2, row2, col2, ew2, zeros2d)

    out = pl.pallas_call(
        _head_body,
        out_shape=jax.ShapeDtypeStruct((N, Wo.shape[1]), jnp.float32),
    )(parts2[0, :N], parts2[1, :N], g2, dis_col, b2.reshape(1, D),
      flat, Wf, bf.reshape(1, -1), Wo[:D], Wo[D:], bo.reshape(1, -1))
    return out


# speedup vs baseline: 11.8765x; 1.1998x over previous
import jax
import jax.numpy as jnp
from jax import lax
from jax.experimental import pallas as pl
from jax.experimental.pallas import tpu as pltpu
from jax.experimental.pallas import tpu_sc as plsc

N = 10000
E = 320000
D = 128
NC = 2
NS = 16
NW = NC * NS
CH = 128
NCH = 80
EPAD = NW * CH * NCH
NPAD = 10240
RPT = NPAD // NS
BLK = 16
NBLK = NCH // BLK

_MESH = plsc.VectorSubcoreMesh(core_axis_name="c", subcore_axis_name="s")


def _sc_deg_body(col2_hbm, ew2_hbm, zn_hbm, degp_hbm, col_v, ew_v, deg_sh):
    c = lax.axis_index("c")
    s = lax.axis_index("s")
    w = s * NC + c
    pltpu.sync_copy(col2_hbm.at[pl.ds(w * NCH, NCH)], col_v)
    pltpu.sync_copy(ew2_hbm.at[pl.ds(w * NCH, NCH)], ew_v)
    pltpu.sync_copy(zn_hbm.at[pl.ds(s * RPT, RPT)],
                    deg_sh.at[pl.ds(s * RPT, RPT)])
    plsc.subcore_barrier()

    def deg_chunk(j, carry):
        pltpu.sync_copy(ew_v.at[j], deg_sh.at[col_v.at[j]], add=True)
        return carry

    lax.fori_loop(0, NCH, deg_chunk, 0)
    plsc.subcore_barrier()
    pltpu.sync_copy(deg_sh.at[pl.ds(s * RPT, RPT)],
                    degp_hbm.at[c, pl.ds(s * RPT, RPT)])


_sc_deg = pl.kernel(
    _sc_deg_body,
    out_type=jax.ShapeDtypeStruct((NC, NPAD), jnp.float32),
    mesh=_MESH,
    compiler_params=pltpu.CompilerParams(needs_layout_passes=False),
    scratch_types=(
        pltpu.VMEM((NCH, CH), jnp.int32),
        pltpu.VMEM((NCH, CH), jnp.float32),
        pltpu.VMEM_SHARED((NPAD,), jnp.float32),
    ),
)


def _sc_prop_body(table_hbm, row2_hbm, col2_hbm, ew2_hbm, z2_hbm,
                  parts_hbm,
                  col_v, ew_v, row_v, w_v, msg0, msg1, mout, acc_sh,
                  sem0, sem1):
    c = lax.axis_index("c")
    s = lax.axis_index("s")
    w = s * NC + c

    pltpu.sync_copy(z2_hbm.at[pl.ds(s * RPT, RPT)],
                    acc_sh.at[pl.ds(s * RPT, RPT)])
    plsc.subcore_barrier()

    def scale_scatter(j, msg):
        for g in range(8):
            sl = pl.ds(g * 16, 16)
            w_v[sl] = ew_v[j, sl]

        def edge_body(e, carry3):
            spl = plsc.load_gather(w_v, [jnp.zeros((16,), jnp.int32) + e])
            for f in range(4):
                v = msg[e, pl.ds(16 * f, 16)]
                vb = plsc.bitcast(v, jnp.bfloat16)
                a, b = plsc.unpack(vb, format=plsc.PackFormat.INTERLEAVED)
                mout[e, pl.ds(32 * f, 16)] = a * spl
                mout[e, pl.ds(32 * f + 16, 16)] = b * spl
            return carry3

        lax.fori_loop(0, CH, edge_body, 0)
        pltpu.sync_copy(mout, acc_sh.at[col_v.at[j]], add=True)

    def block_body(b, carry):
        base = w * NCH + b * BLK
        pltpu.sync_copy(col2_hbm.at[pl.ds(base, BLK)], col_v)
        pltpu.sync_copy(ew2_hbm.at[pl.ds(base, BLK)], ew_v)
        pltpu.sync_copy(row2_hbm.at[pl.ds(base, BLK)], row_v)
        pltpu.async_copy(table_hbm.at[row_v.at[0]], msg0, sem0)

        def pair_body(p, carry2):
            ja = 2 * p
            jb = 2 * p + 1
            pltpu.make_async_copy(table_hbm.at[row_v.at[ja]], msg0, sem0).wait()
            pltpu.async_copy(table_hbm.at[row_v.at[jb]], msg1, sem1)
            scale_scatter(ja, msg0)
            pltpu.make_async_copy(table_hbm.at[row_v.at[jb]], msg1, sem1).wait()

            @pl.when(p < BLK // 2 - 1)
            def _():
                pltpu.async_copy(table_hbm.at[row_v.at[ja + 2]], msg0, sem0)

            scale_scatter(jb, msg1)
            return carry2

        lax.fori_loop(0, BLK // 2, pair_body, 0)
        return carry

    lax.fori_loop(0, NBLK, block_body, 0)
    plsc.subcore_barrier()
    pltpu.sync_copy(acc_sh.at[pl.ds(s * RPT, RPT)],
                    parts_hbm.at[c, pl.ds(s * RPT, RPT)])


_sc_prop = pl.kernel(
    _sc_prop_body,
    out_type=jax.ShapeDtypeStruct((NC, NPAD, D), jnp.float32),
    mesh=_MESH,
    compiler_params=pltpu.CompilerParams(needs_layout_passes=False,
                                         use_tc_tiling_on_sc=False),
    scratch_types=(
        pltpu.VMEM((BLK, CH), jnp.int32),
        pltpu.VMEM((BLK, CH), jnp.float32),
        pltpu.VMEM((BLK, CH), jnp.int32),
        pltpu.VMEM((CH,), jnp.float32),
        pltpu.VMEM((CH, D // 2), jnp.float32),
        pltpu.VMEM((CH, D // 2), jnp.float32),
        pltpu.VMEM((CH, D), jnp.float32),
        pltpu.VMEM_SHARED((NPAD, D), jnp.float32),
        pltpu.SemaphoreType.DMA,
        pltpu.SemaphoreType.DMA,
    ),
)


def _first_body(x_ref, w1_ref, p0_ref, p1_ref, g1_ref, g1b_ref, dis_ref):
    dis = lax.rsqrt(1.0 + p0_ref[...] + p1_ref[...])
    dis_ref[...] = dis
    g = dis[:N] * jnp.dot(x_ref[...], w1_ref[...],
                          preferred_element_type=jnp.float32)
    g1_ref[...] = g
    g1b_ref[...] = g.astype(jnp.bfloat16)


def _mid_body(r0_ref, r1_ref, g1_ref, dis_ref, b1_ref, w2_ref, perm_ref,
              o_ref, ob_ref):
    dis = dis_ref[...]
    r = jnp.dot(r0_ref[...] + r1_ref[...], perm_ref[...],
                preferred_element_type=jnp.float32)
    t = dis * (r + g1_ref[...]) + b1_ref[...]
    t = jnp.maximum(t, 0.0)
    g = dis * jnp.dot(t, w2_ref[...], preferred_element_type=jnp.float32)
    o_ref[...] = g
    ob_ref[...] = g.astype(jnp.bfloat16)


def _head_body(r0_ref, r1_ref, g2_ref, dis_ref, b2_ref, flat_ref, wf_ref,
               bf_ref, woh_ref, wof_ref, bo_ref, perm_ref, o_ref):
    dis = dis_ref[...]
    r = jnp.dot(r0_ref[...] + r1_ref[...], perm_ref[...],
                preferred_element_type=jnp.float32)
    z = dis * (r + g2_ref[...]) + b2_ref[...]
    f = jnp.dot(flat_ref[...], wf_ref[...],
                preferred_element_type=jnp.float32) + bf_ref[...]
    out = (jnp.dot(z, woh_ref[...], preferred_element_type=jnp.float32)
           + jnp.dot(f, wof_ref[...], preferred_element_type=jnp.float32)
           + bo_ref[...])
    o_ref[...] = jnp.maximum(out, 0.0)


def kernel(x, flat, edge_index, edge_weight, W1, b1, W2, b2, Wf, bf, Wo, bo):
    row = edge_index[0].astype(jnp.int32)
    col = edge_index[1].astype(jnp.int32)
    ew = edge_weight.reshape(-1).astype(jnp.float32)
    pad = EPAD - E
    row2 = jnp.concatenate([row, jnp.zeros((pad,), jnp.int32)]).reshape(-1, CH)
    col2 = jnp.concatenate([col, jnp.zeros((pad,), jnp.int32)]).reshape(-1, CH)
    ew2 = jnp.concatenate([ew, jnp.zeros((pad,), jnp.float32)]).reshape(-1, CH)
    zeros2d = jnp.zeros((NPAD, D), jnp.float32)
    zerosn = jnp.zeros((NPAD,), jnp.float32)

    pi = jnp.asarray(
        [32 * (i // 32) + (2 * (i % 32) if i % 32 < 16 else 2 * (i % 32 - 16) + 1)
         for i in range(D)], dtype=jnp.int32)
    P = jnp.zeros((D, D), jnp.float32).at[jnp.arange(D), pi].set(1.0)

    degp = _sc_deg(col2, ew2, zerosn)

    g1, g1b, dis = pl.pallas_call(
        _first_body,
        out_shape=(jax.ShapeDtypeStruct((N, D), jnp.float32),
                   jax.ShapeDtypeStruct((N, D), jnp.bfloat16),
                   jax.ShapeDtypeStruct((NPAD, 1), jnp.float32)),
    )(x, W1, degp[0].reshape(NPAD, 1), degp[1].reshape(NPAD, 1))

    g1v = lax.bitcast_convert_type(g1b.reshape(N, D // 2, 2), jnp.float32)
    parts1 = _sc_prop(g1v, row2, col2, ew2, zeros2d)

    dis_col = dis[:N]
    g2, g2b = pl.pallas_call(
        _mid_body,
        out_shape=(jax.ShapeDtypeStruct((N, D), jnp.float32),
                   jax.ShapeDtypeStruct((N, D), jnp.bfloat16)),
    )(parts1[0, :N], parts1[1, :N], g1, dis_col, b1.reshape(1, D), W2, P)

    g2v = lax.bitcast_convert_type(g2b.reshape(N, D // 2, 2), jnp.float32)
    parts2 = _sc_prop(g2v, row2, col2, ew2, zeros2d)

    out = pl.pallas_call(
        _head_body,
        out_shape=jax.ShapeDtypeStruct((N, Wo.shape[1]), jnp.float32),
    )(parts2[0, :N], parts2[1, :N], g2, dis_col, b2.reshape(1, D),
      flat, Wf, bf.reshape(1, -1), Wo[:D], Wo[D:], bo.reshape(1, -1), P)
    return out


# speedup vs baseline: 14.0711x; 1.1848x over previous
import jax
import jax.numpy as jnp
from jax import lax
from jax.experimental import pallas as pl
from jax.experimental.pallas import tpu as pltpu
from jax.experimental.pallas import tpu_sc as plsc

N = 10000
E = 320000
D = 128
NC = 2
NS = 16
NW = NC * NS
CH = 128
NCH = 80
EPAD = NW * CH * NCH
NPAD = 10240
RPT = NPAD // NS
BLK = 16
NBLK = NCH // BLK

_MESH = plsc.VectorSubcoreMesh(core_axis_name="c", subcore_axis_name="s")


def _sc_deg_body(col2_hbm, ew2_hbm, zn_hbm, degp_hbm, col_v, ew_v, deg_sh):
    c = lax.axis_index("c")
    s = lax.axis_index("s")
    w = s * NC + c
    pltpu.sync_copy(col2_hbm.at[pl.ds(w * NCH, NCH)], col_v)
    pltpu.sync_copy(ew2_hbm.at[pl.ds(w * NCH, NCH)], ew_v)
    pltpu.sync_copy(zn_hbm.at[pl.ds(s * RPT, RPT)],
                    deg_sh.at[pl.ds(s * RPT, RPT)])
    plsc.subcore_barrier()

    def deg_chunk(j, carry):
        pltpu.sync_copy(ew_v.at[j], deg_sh.at[col_v.at[j]], add=True)
        return carry

    lax.fori_loop(0, NCH, deg_chunk, 0)
    plsc.subcore_barrier()
    pltpu.sync_copy(deg_sh.at[pl.ds(s * RPT, RPT)],
                    degp_hbm.at[c, pl.ds(s * RPT, RPT)])


_sc_deg = pl.kernel(
    _sc_deg_body,
    out_type=jax.ShapeDtypeStruct((NC, NPAD), jnp.float32),
    mesh=_MESH,
    compiler_params=pltpu.CompilerParams(needs_layout_passes=False),
    scratch_types=(
        pltpu.VMEM((NCH, CH), jnp.int32),
        pltpu.VMEM((NCH, CH), jnp.float32),
        pltpu.VMEM_SHARED((NPAD,), jnp.float32),
    ),
)


def _sc_prop_body(table_hbm, row2_hbm, col2_hbm, ew2_hbm, z2_hbm,
                  parts_hbm,
                  col_v, ew_v, row_v, w_v, msg0, msg1, mout, acc_sh,
                  sem0, sem1):
    c = lax.axis_index("c")
    s = lax.axis_index("s")
    w = s * NC + c

    pltpu.sync_copy(z2_hbm.at[pl.ds(s * RPT, RPT)],
                    acc_sh.at[pl.ds(s * RPT, RPT)])
    plsc.subcore_barrier()

    def scale_scatter(j, msg):
        for g in range(8):
            sl = pl.ds(g * 16, 16)
            w_v[sl] = ew_v[j, sl]

        @plsc.parallel_loop(0, CH, 1, unroll=4)
        def _(e):
            spl = plsc.load_gather(w_v, [jnp.zeros((16,), jnp.int32) + e])
            for f in range(4):
                v = msg[e, pl.ds(16 * f, 16)]
                vb = plsc.bitcast(v, jnp.bfloat16)
                a, b = plsc.unpack(vb, format=plsc.PackFormat.INTERLEAVED)
                mout[e, pl.ds(32 * f, 16)] = a * spl
                mout[e, pl.ds(32 * f + 16, 16)] = b * spl
        pltpu.sync_copy(mout, acc_sh.at[col_v.at[j]], add=True)

    def block_body(b, carry):
        base = w * NCH + b * BLK
        pltpu.sync_copy(col2_hbm.at[pl.ds(base, BLK)], col_v)
        pltpu.sync_copy(ew2_hbm.at[pl.ds(base, BLK)], ew_v)
        pltpu.sync_copy(row2_hbm.at[pl.ds(base, BLK)], row_v)
        pltpu.async_copy(table_hbm.at[row_v.at[0]], msg0, sem0)

        def pair_body(p, carry2):
            ja = 2 * p
            jb = 2 * p + 1
            pltpu.make_async_copy(table_hbm.at[row_v.at[ja]], msg0, sem0).wait()
            pltpu.async_copy(table_hbm.at[row_v.at[jb]], msg1, sem1)
            scale_scatter(ja, msg0)
            pltpu.make_async_copy(table_hbm.at[row_v.at[jb]], msg1, sem1).wait()

            @pl.when(p < BLK // 2 - 1)
            def _():
                pltpu.async_copy(table_hbm.at[row_v.at[ja + 2]], msg0, sem0)

            scale_scatter(jb, msg1)
            return carry2

        lax.fori_loop(0, BLK // 2, pair_body, 0)
        return carry

    lax.fori_loop(0, NBLK, block_body, 0)
    plsc.subcore_barrier()
    pltpu.sync_copy(acc_sh.at[pl.ds(s * RPT, RPT)],
                    parts_hbm.at[c, pl.ds(s * RPT, RPT)])


_sc_prop = pl.kernel(
    _sc_prop_body,
    out_type=jax.ShapeDtypeStruct((NC, NPAD, D), jnp.float32),
    mesh=_MESH,
    compiler_params=pltpu.CompilerParams(needs_layout_passes=False,
                                         use_tc_tiling_on_sc=False),
    scratch_types=(
        pltpu.VMEM((BLK, CH), jnp.int32),
        pltpu.VMEM((BLK, CH), jnp.float32),
        pltpu.VMEM((BLK, CH), jnp.int32),
        pltpu.VMEM((CH,), jnp.float32),
        pltpu.VMEM((CH, D // 2), jnp.float32),
        pltpu.VMEM((CH, D // 2), jnp.float32),
        pltpu.VMEM((CH, D), jnp.float32),
        pltpu.VMEM_SHARED((NPAD, D), jnp.float32),
        pltpu.SemaphoreType.DMA,
        pltpu.SemaphoreType.DMA,
    ),
)


def _first_body(x_ref, w1_ref, p0_ref, p1_ref, g1_ref, g1b_ref, dis_ref):
    dis = lax.rsqrt(1.0 + p0_ref[...] + p1_ref[...])
    dis_ref[...] = dis
    g = dis[:N] * jnp.dot(x_ref[...], w1_ref[...],
                          preferred_element_type=jnp.float32)
    g1_ref[...] = g
    g1b_ref[...] = g.astype(jnp.bfloat16)


def _mid_body(r0_ref, r1_ref, g1_ref, dis_ref, b1_ref, w2_ref, perm_ref,
              o_ref, ob_ref):
    dis = dis_ref[...]
    r = jnp.dot(r0_ref[...] + r1_ref[...], perm_ref[...],
                preferred_element_type=jnp.float32)
    t = dis * (r + g1_ref[...]) + b1_ref[...]
    t = jnp.maximum(t, 0.0)
    g = dis * jnp.dot(t, w2_ref[...], preferred_element_type=jnp.float32)
    o_ref[...] = g
    ob_ref[...] = g.astype(jnp.bfloat16)


def _head_body(r0_ref, r1_ref, g2_ref, dis_ref, b2_ref, flat_ref, wf_ref,
               bf_ref, woh_ref, wof_ref, bo_ref, perm_ref, o_ref):
    dis = dis_ref[...]
    r = jnp.dot(r0_ref[...] + r1_ref[...], perm_ref[...],
                preferred_element_type=jnp.float32)
    z = dis * (r + g2_ref[...]) + b2_ref[...]
    f = jnp.dot(flat_ref[...], wf_ref[...],
                preferred_element_type=jnp.float32) + bf_ref[...]
    out = (jnp.dot(z, woh_ref[...], preferred_element_type=jnp.float32)
           + jnp.dot(f, wof_ref[...], preferred_element_type=jnp.float32)
           + bo_ref[...])
    o_ref[...] = jnp.maximum(out, 0.0)


def kernel(x, flat, edge_index, edge_weight, W1, b1, W2, b2, Wf, bf, Wo, bo):
    row = edge_index[0].astype(jnp.int32)
    col = edge_index[1].astype(jnp.int32)
    ew = edge_weight.reshape(-1).astype(jnp.float32)
    pad = EPAD - E
    row2 = jnp.concatenate([row, jnp.zeros((pad,), jnp.int32)]).reshape(-1, CH)
    col2 = jnp.concatenate([col, jnp.zeros((pad,), jnp.int32)]).reshape(-1, CH)
    ew2 = jnp.concatenate([ew, jnp.zeros((pad,), jnp.float32)]).reshape(-1, CH)
    zeros2d = jnp.zeros((NPAD, D), jnp.float32)
    zerosn = jnp.zeros((NPAD,), jnp.float32)

    pi = jnp.asarray(
        [32 * (i // 32) + (2 * (i % 32) if i % 32 < 16 else 2 * (i % 32 - 16) + 1)
         for i in range(D)], dtype=jnp.int32)
    P = jnp.zeros((D, D), jnp.float32).at[jnp.arange(D), pi].set(1.0)

    degp = _sc_deg(col2, ew2, zerosn)

    g1, g1b, dis = pl.pallas_call(
        _first_body,
        out_shape=(jax.ShapeDtypeStruct((N, D), jnp.float32),
                   jax.ShapeDtypeStruct((N, D), jnp.bfloat16),
                   jax.ShapeDtypeStruct((NPAD, 1), jnp.float32)),
    )(x, W1, degp[0].reshape(NPAD, 1), degp[1].reshape(NPAD, 1))

    g1v = lax.bitcast_convert_type(g1b.reshape(N, D // 2, 2), jnp.float32)
    parts1 = _sc_prop(g1v, row2, col2, ew2, zeros2d)

    dis_col = dis[:N]
    g2, g2b = pl.pallas_call(
        _mid_body,
        out_shape=(jax.ShapeDtypeStruct((N, D), jnp.float32),
                   jax.ShapeDtypeStruct((N, D), jnp.bfloat16)),
    )(parts1[0, :N], parts1[1, :N], g1, dis_col, b1.reshape(1, D), W2, P)

    g2v = lax.bitcast_convert_type(g2b.reshape(N, D // 2, 2), jnp.float32)
    parts2 = _sc_prop(g2v, row2, col2, ew2, zeros2d)

    out = pl.pallas_call(
        _head_body,
        out_shape=jax.ShapeDtypeStruct((N, Wo.shape[1]), jnp.float32),
    )(parts2[0, :N], parts2[1, :N], g2, dis_col, b2.reshape(1, D),
      flat, Wf, bf.reshape(1, -1), Wo[:D], Wo[D:], bo.reshape(1, -1), P)
    return out
